# baseline (speedup 1.0000x reference)
import functools

import jax
import jax.numpy as jnp
from jax import lax
from jax.experimental import pallas as pl
from jax.experimental.pallas import tpu as pltpu, tpu_sc as plsc

BATCH = 4096
HIST = 200
D = 128
B = BATCH * HIST
ROWS = B // 128
NC = 2
NS = 16
NW = NC * NS
ROWS_PER_W = ROWS // NW
NCH = ROWS_PER_W
NBUF = 5


def _make_emb():
    mesh = plsc.VectorSubcoreMesh(core_axis_name="c", subcore_axis_name="s")

    @functools.partial(
        pl.kernel,
        mesh=mesh,
        out_type=jax.ShapeDtypeStruct((ROWS, 128, D), jnp.float32),
        scratch_types=[
            pltpu.VMEM((ROWS_PER_W, 128), jnp.int32),
            pltpu.VMEM((NBUF, 128, D), jnp.float32),
            pltpu.VMEM_SHARED((1000, D), jnp.float32),
            pltpu.SemaphoreType.DMA,
            pltpu.SemaphoreType.DMA,
            pltpu.SemaphoreType.DMA,
            pltpu.SemaphoreType.DMA,
            pltpu.SemaphoreType.DMA,
            pltpu.SemaphoreType.DMA,
            pltpu.SemaphoreType.DMA,
            pltpu.SemaphoreType.DMA,
            pltpu.SemaphoreType.DMA,
            pltpu.SemaphoreType.DMA,
            pltpu.SemaphoreType.DMA,
        ],
    )
    def emb(x_hbm, w_hbm, out_hbm, idx_v, rows_v, w_sp,
            sg0, sg1, sg2, sg3, sg4, ss0, ss1, ss2, ss3, ss4, si):
        wid = lax.axis_index("s") * NC + lax.axis_index("c")
        row0 = wid * ROWS_PER_W
        sems_g = (sg0, sg1, sg2, sg3, sg4)
        sems_s = (ss0, ss1, ss2, ss3, ss4)

        idx_cp = pltpu.async_copy(x_hbm.at[pl.ds(row0, ROWS_PER_W)], idx_v, si)

        @pl.when(lax.axis_index("s") == 0)
        def _():
            pltpu.sync_copy(w_hbm, w_sp)

        idx_cp.wait()
        plsc.subcore_barrier()

        def fire_gather(c, b):
            pltpu.async_copy(w_sp.at[idx_v.at[c]], rows_v.at[b], sems_g[b])

        def wait_gather(b):
            pltpu.make_async_copy(
                w_sp.at[idx_v.at[0]], rows_v.at[b], sems_g[b]).wait()

        def fire_scatter(c, b):
            pltpu.async_copy(rows_v.at[b], out_hbm.at[row0 + c], sems_s[b])

        def wait_scatter(b):
            pltpu.make_async_copy(
                rows_v.at[b], out_hbm.at[row0], sems_s[b]).wait()

        def pairstep(c, cb, fire):
            wait_gather(cb % NBUF)
            wait_gather((cb + 1) % NBUF)
            fire_scatter(c, cb % NBUF)
            fire_scatter(c + 1, (cb + 1) % NBUF)
            wait_scatter((cb + 3) % NBUF)
            wait_scatter((cb + 4) % NBUF)
            if fire >= 1:
                fire_gather(c + 3, (cb + 3) % NBUF)
            if fire >= 2:
                fire_gather(c + 4, (cb + 4) % NBUF)

        fire_gather(0, 0)
        fire_gather(1, 1)
        fire_gather(2, 2)
        wait_gather(0)
        wait_gather(1)
        fire_scatter(0, 0)
        fire_scatter(1, 1)
        fire_gather(3, 3)
        fire_gather(4, 4)

        def group(g, carry):
            c0 = 10 * g + 2
            for i in range(5):
                pairstep(c0 + 2 * i, (2 + 2 * i) % NBUF, 2)
            return carry

        lax.fori_loop(0, (NCH - 10) // 10, group, 0)

        pairstep(NCH - 8, (NCH - 8) % NBUF, 2)
        pairstep(NCH - 6, (NCH - 6) % NBUF, 2)
        pairstep(NCH - 4, (NCH - 4) % NBUF, 1)
        pairstep(NCH - 2, (NCH - 2) % NBUF, 0)
        wait_scatter((NCH - 2) % NBUF)
        wait_scatter((NCH - 1) % NBUF)

    return emb


_emb = _make_emb()


def kernel(x, W):
    x2 = x.reshape(ROWS, 128)
    out = _emb(x2, W)
    return out.reshape(BATCH, HIST, D)
